# baseline (speedup 1.0000x reference)
import functools

import jax
import jax.numpy as jnp
from jax import lax
from jax.experimental import pallas as pl
from jax.experimental.pallas import tpu as pltpu

NUM_EXPERTS = 64
TOP_K = 8
MODEL_DIM = 2048
T = 16384
BLOCK_T = 1024


def _router_block(x_ref, w_ref, probs_ref, tw_ref, ti_ref):
    x = x_ref[...]
    w = w_ref[...]
    logits_t = lax.dot_general(
        w, x,
        dimension_numbers=(((1,), (1,)), ((), ())),
        preferred_element_type=jnp.float32,
    )
    m = jnp.max(logits_t, axis=0, keepdims=True)
    e = jnp.exp(logits_t - m)
    s = jnp.sum(e, axis=0, keepdims=True)
    probs_ref[...] = (e * (1.0 / s)).T

    iota_e = lax.broadcasted_iota(jnp.int32, e.shape, 0)
    p = e
    vrows, irows = [], []
    for _ in range(TOP_K):
        cur = jnp.max(p, axis=0, keepdims=True)
        idx = jnp.min(jnp.where(p == cur, iota_e, NUM_EXPERTS), axis=0,
                      keepdims=True)
        vrows.append(cur)
        irows.append(idx)
        p = jnp.where(iota_e == idx, -1.0, p)
    vals = jnp.concatenate(vrows, axis=0)
    tw_ref[...] = vals * (1.0 / jnp.sum(vals, axis=0, keepdims=True))
    ti_ref[...] = jnp.concatenate(irows, axis=0)


@functools.partial(jax.jit, static_argnames=("interpret",))
def _run(hidden_states, weight, interpret=False):
    x = hidden_states.reshape(-1, MODEL_DIM)
    grid = (T // BLOCK_T,)
    probs, tw_t, ti_t = pl.pallas_call(
        _router_block,
        grid=grid,
        in_specs=[
            pl.BlockSpec((BLOCK_T, MODEL_DIM), lambda i: (i, 0)),
            pl.BlockSpec((NUM_EXPERTS, MODEL_DIM), lambda i: (0, 0)),
        ],
        out_specs=[
            pl.BlockSpec((BLOCK_T, NUM_EXPERTS), lambda i: (i, 0)),
            pl.BlockSpec((TOP_K, BLOCK_T), lambda i: (0, i)),
            pl.BlockSpec((TOP_K, BLOCK_T), lambda i: (0, i)),
        ],
        out_shape=[
            jax.ShapeDtypeStruct((T, NUM_EXPERTS), jnp.float32),
            jax.ShapeDtypeStruct((TOP_K, T), jnp.float32),
            jax.ShapeDtypeStruct((TOP_K, T), jnp.int32),
        ],
        interpret=interpret,
    )(x, weight)
    return probs, tw_t.T, ti_t.T


def kernel(hidden_states, weight):
    return _run(hidden_states, weight)


# speedup vs baseline: 1.0019x; 1.0019x over previous
import functools

import jax
import jax.numpy as jnp
from jax import lax
from jax.experimental import pallas as pl

NUM_EXPERTS = 64
TOP_K = 8
MODEL_DIM = 2048
T = 16384
BLOCK_T = 1024


def _router_block(x_ref, w_ref, probs_ref, tw_ref, ti_ref):
    x = x_ref[...]
    w = w_ref[...]
    logits_t = lax.dot_general(
        w, x,
        dimension_numbers=(((1,), (1,)), ((), ())),
        preferred_element_type=jnp.float32,
    )
    m = jnp.max(logits_t, axis=0, keepdims=True)
    e = jnp.exp(logits_t - m)
    s = jnp.sum(e, axis=0, keepdims=True)
    probs_ref[...] = (e * (1.0 / s)).T

    iota_e = lax.broadcasted_iota(jnp.int32, e.shape, 0)
    p = e
    vrows, irows = [], []
    for _ in range(TOP_K):
        cur = jnp.max(p, axis=0, keepdims=True)
        idx = jnp.min(jnp.where(p == cur, iota_e, NUM_EXPERTS), axis=0,
                      keepdims=True)
        vrows.append(cur)
        irows.append(idx)
        p = jnp.where(iota_e == idx, -1.0, p)
    vals = jnp.concatenate(vrows, axis=0)
    tw_ref[...] = vals * (1.0 / jnp.sum(vals, axis=0, keepdims=True))
    ti_ref[...] = jnp.concatenate(irows, axis=0)


@functools.partial(jax.jit, static_argnames=("interpret",))
def _run(hidden_states, weight, interpret=False):
    x = hidden_states.reshape(-1, MODEL_DIM)
    grid = (T // BLOCK_T,)
    probs, tw_t, ti_t = pl.pallas_call(
        _router_block,
        grid=grid,
        in_specs=[
            pl.BlockSpec((BLOCK_T, MODEL_DIM), lambda i: (i, 0)),
            pl.BlockSpec((NUM_EXPERTS, MODEL_DIM), lambda i: (0, 0)),
        ],
        out_specs=[
            pl.BlockSpec((BLOCK_T, NUM_EXPERTS), lambda i: (i, 0)),
            pl.BlockSpec((TOP_K, BLOCK_T), lambda i: (0, i)),
            pl.BlockSpec((TOP_K, BLOCK_T), lambda i: (0, i)),
        ],
        out_shape=[
            jax.ShapeDtypeStruct((T, NUM_EXPERTS), jnp.float32),
            jax.ShapeDtypeStruct((TOP_K, T), jnp.float32),
            jax.ShapeDtypeStruct((TOP_K, T), jnp.int32),
        ],
        interpret=interpret,
    )(x, weight)
    return probs, tw_t.T, ti_t.T


def kernel(hidden_states, weight):
    return _run(hidden_states, weight)
